# baseline (speedup 1.0000x reference)
import functools

import jax
import jax.numpy as jnp
from jax import lax
from jax.experimental import pallas as pl
from jax.experimental.pallas import tpu as pltpu
from jax.experimental.pallas import tpu_sc as plsc

N = 10000
E = 320000
IN_D = 128
HID = 32
HEADS = 8
OUT_D = 64
NEG = 0.2

NC, NS, LANES = 2, 16, 16
ET = E + N
SB = 128
EPR = 2688
EP = EPR * 128
NP = 10240
RW = 80
ROWS_PT = NP // NS
ZB = 32

NB1 = EPR // NS
NB2 = EPR // (NC * NS)

_mesh = plsc.VectorSubcoreMesh(core_axis_name="c", subcore_axis_name="s")


def _take16(v, idx):
    dn = lax.GatherDimensionNumbers(
        offset_dims=(), collapsed_slice_dims=(0,), start_index_map=(0,))
    return lax.gather(v, idx[:, None], dn, (1,),
                      mode=lax.GatherScatterMode.PROMISE_IN_BOUNDS)


def _k1_body(x_ref, w1_ref, attl_ref, attr_ref, ms_ref, tab_ref, s_ref,
             vmax):
    i = pl.program_id(0)
    qq = pl.program_id(1)
    h = jnp.dot(x_ref[...], w1_ref[...], preferred_element_type=jnp.float32)
    tab_ref[...] = jnp.dot(h, ms_ref[0], preferred_element_type=jnp.float32)

    @pl.when(qq == 0)
    def _():
        al = jnp.dot(h, attl_ref[...], preferred_element_type=jnp.float32)
        ar = jnp.dot(h, attr_ref[...], preferred_element_type=jnp.float32)

        @pl.when(i == 0)
        def _():
            vmax[...] = jnp.full((8, 128), -jnp.inf, jnp.float32)

        vmax[0:1, 0:8] = jnp.maximum(
            vmax[0:1, 0:8], jnp.max(al, axis=0, keepdims=True))
        vmax[0:1, 8:16] = jnp.maximum(
            vmax[0:1, 8:16], jnp.max(ar, axis=0, keepdims=True))

    @pl.when((i == pl.num_programs(0) - 1) & (qq == pl.num_programs(1) - 1))
    def _():
        for hh in range(8):
            s_ref[hh] = jnp.maximum(vmax[0, hh] + vmax[0, 8 + hh], 0.0)
        for k in range(8, 16):
            s_ref[k] = 0.0


def _tc_layer1(x, W1, attl_mat, attr_mat, ms):
    bn = 2000
    grid = N // bn
    return pl.pallas_call(
        _k1_body,
        grid=(grid, 4),
        in_specs=[
            pl.BlockSpec((bn, IN_D), lambda i, qq: (i, 0)),
            pl.BlockSpec((IN_D, HEADS * HID), lambda i, qq: (0, 0)),
            pl.BlockSpec((HEADS * HID, 8), lambda i, qq: (0, 0)),
            pl.BlockSpec((HEADS * HID, 8), lambda i, qq: (0, 0)),
            pl.BlockSpec((1, HEADS * HID, RW), lambda i, qq: (qq, 0, 0)),
        ],
        out_specs=[
            pl.BlockSpec((bn, RW), lambda i, qq: (qq * (N // bn) + i, 0)),
            pl.BlockSpec(memory_space=pltpu.SMEM),
        ],
        out_shape=[
            jax.ShapeDtypeStruct((4 * N, RW), jnp.float32),
            jax.ShapeDtypeStruct((16,), jnp.float32),
        ],
        scratch_shapes=[pltpu.VMEM((8, 128), jnp.float32)],
    )(x, W1, attl_mat, attr_mat, ms)


def _zero_acc(zbuf, acc, s):
    zero16 = jnp.zeros((16,), jnp.float32)
    for r in range(ZB):
        for k in range(RW // 16):
            zbuf[r, pl.ds(16 * k, 16)] = zero16
    for j in range(ROWS_PT // ZB):
        pltpu.sync_copy(zbuf, acc.at[pl.ds(s * ROWS_PT + j * ZB, ZB)])


def _fire(idx_slice, ibuf, tab_hbm, srows, drows, gsem, qoff=None):
    pltpu.sync_copy(idx_slice, ibuf)
    if qoff is not None:
        for j in range(2):
            for k in range(8):
                sl = pl.ds(16 * k, 16)
                ibuf[0, j, sl] = ibuf[0, j, sl] + qoff
    pltpu.async_copy(tab_hbm.at[ibuf.at[0, 0]], srows, gsem)
    pltpu.async_copy(tab_hbm.at[ibuf.at[0, 1]], drows, gsem)


def _consume(ibuf, tab_hbm, srows, drows, stage, acc, gsem, run_edges):
    pltpu.make_async_copy(tab_hbm.at[ibuf.at[0, 0]], srows, gsem).wait()
    pltpu.make_async_copy(tab_hbm.at[ibuf.at[0, 1]], drows, gsem).wait()
    run_edges()
    pltpu.sync_copy(stage, acc.at[ibuf.at[0, 2]], add=True)


def _sc1_body(tab_hbm, idx_hbm, s_hbm, out_hbm,
              ibuf0, ibuf1, srows0, drows0, srows1, drows1, stage0,
              zbuf, svec, acc, gsem0, gsem1):
    c = lax.axis_index("c")
    s = lax.axis_index("s")
    iota = lax.iota(jnp.int32, 16)
    sh2 = jnp.where(iota < 2, iota + 2, iota)
    par = iota & 1
    base128 = s * (EPR // NS)

    for p in range(2):
        q = 2 * c + p
        qoff = q * N
        _zero_acc(zbuf, acc, s)
        pltpu.sync_copy(s_hbm.at[q], svec)
        plsc.subcore_barrier()
        sv = svec[...]

        def make_run_edges(srows, drows, stage):
            def run_edges():
                @plsc.parallel_loop(0, SB, unroll=8)
                def _loop(e):
                    s0 = srows[e, pl.ds(0, 16)]
                    s1 = srows[e, pl.ds(16, 16)]
                    s2 = srows[e, pl.ds(32, 16)]
                    s3 = srows[e, pl.ds(48, 16)]
                    m = (s0 * drows[e, pl.ds(0, 16)]
                         + s1 * drows[e, pl.ds(16, 16)]
                         + s2 * drows[e, pl.ds(32, 16)]
                         + s3 * drows[e, pl.ds(48, 16)])
                    for sh in (8, 4, 2):
                        m = m + _take16(m, iota ^ sh)
                    ts = srows[e, pl.ds(64, 16)]
                    td = drows[e, pl.ds(64, 16)]
                    base = ts + _take16(td, sh2)
                    sig = 1.0 / (1.0 + jnp.exp(-m))
                    a0 = base * sig
                    alpha = jnp.where(a0 >= 0, a0, NEG * a0)
                    w = jnp.exp(alpha - sv)
                    wmix = _take16(w, par)
                    stage[e, pl.ds(0, 16)] = s0 * wmix
                    stage[e, pl.ds(16, 16)] = s1 * wmix
                    stage[e, pl.ds(32, 16)] = s2 * wmix
                    stage[e, pl.ds(48, 16)] = s3 * wmix
                    stage[e, pl.ds(64, 16)] = w
            return run_edges

        eb0 = make_run_edges(srows0, drows0, stage0)
        eb1 = make_run_edges(srows1, drows1, stage0)

        def fire(sb, ibuf, srows, drows, gsem):
            _fire(idx_hbm.at[pl.ds(base128 + sb, 1)], ibuf,
                  tab_hbm, srows, drows, gsem, qoff=qoff)

        fire(0, ibuf0, srows0, drows0, gsem0)
        fire(1, ibuf1, srows1, drows1, gsem1)

        def pair_body(k, _):
            _consume(ibuf0, tab_hbm, srows0, drows0, stage0, acc, gsem0, eb0)
            fire(2 * k + 2, ibuf0, srows0, drows0, gsem0)
            _consume(ibuf1, tab_hbm, srows1, drows1, stage0, acc, gsem1, eb1)
            fire(2 * k + 3, ibuf1, srows1, drows1, gsem1)
            return 0

        lax.fori_loop(0, NB1 // 2 - 1, pair_body, 0)
        _consume(ibuf0, tab_hbm, srows0, drows0, stage0, acc, gsem0, eb0)
        _consume(ibuf1, tab_hbm, srows1, drows1, stage0, acc, gsem1, eb1)
        plsc.subcore_barrier()
        pltpu.sync_copy(acc.at[pl.ds(s * ROWS_PT, ROWS_PT)],
                        out_hbm.at[q, pl.ds(s * ROWS_PT, ROWS_PT)])
        plsc.subcore_barrier()


_sc_scratch = [
    pltpu.VMEM((1, 3, 128), jnp.int32),
    pltpu.VMEM((1, 3, 128), jnp.int32),
    pltpu.VMEM((SB, RW), jnp.float32),
    pltpu.VMEM((SB, RW), jnp.float32),
    pltpu.VMEM((SB, RW), jnp.float32),
    pltpu.VMEM((SB, RW), jnp.float32),
    pltpu.VMEM((SB, RW), jnp.float32),
    pltpu.VMEM((ZB, RW), jnp.float32),
    pltpu.VMEM((16,), jnp.float32),
    pltpu.VMEM_SHARED((NP, RW), jnp.float32),
    pltpu.SemaphoreType.DMA,
    pltpu.SemaphoreType.DMA,
]

_sc1_call = functools.partial(
    pl.kernel,
    out_type=jax.ShapeDtypeStruct((4, NP, RW), jnp.float32),
    mesh=_mesh,
    compiler_params=pltpu.CompilerParams(use_tc_tiling_on_sc=False),
    scratch_types=_sc_scratch,
)(_sc1_body)


def _k2_body(acc_ref, m2_ref, b1_ref, r2_ref, p64t_ref, tab_ref, s_ref,
             vmax):
    i = pl.program_id(0)
    parts = []
    for q in range(4):
        aq = acc_ref[q, :, :]
        fde = jnp.dot(aq[:, :64], p64t_ref[...],
                      preferred_element_type=jnp.float32)
        rec = jnp.dot(1.0 / (aq[:, 64:66] + 1e-16), r2_ref[...],
                      preferred_element_type=jnp.float32)
        parts.append(fde * rec)
    out1 = jnp.concatenate(parts, axis=1) + b1_ref[...]
    act = jnp.where(out1 > 0, out1, jnp.exp(jnp.minimum(out1, 0.0)) - 1.0)
    tab = jnp.dot(act, m2_ref[...], preferred_element_type=jnp.float32)
    tab_ref[...] = tab

    @pl.when(i == 0)
    def _():
        vmax[...] = jnp.full((8, 128), -jnp.inf, jnp.float32)

    vmax[0:1, 0:2] = jnp.maximum(
        vmax[0:1, 0:2], jnp.max(tab[:, 64:66], axis=0, keepdims=True))

    @pl.when(i == pl.num_programs(0) - 1)
    def _():
        s_ref[0] = jnp.maximum(vmax[0, 0] + vmax[0, 1], 0.0)
        s_ref[1] = 0.0


def _tc_layer2(acc1, m2, b1, r2, p64t):
    bn = 2000
    grid = N // bn
    return pl.pallas_call(
        _k2_body,
        grid=(grid,),
        in_specs=[
            pl.BlockSpec((4, bn, RW), lambda i: (0, i, 0)),
            pl.BlockSpec((HEADS * HID, RW), lambda i: (0, 0)),
            pl.BlockSpec((1, HEADS * HID), lambda i: (0, 0)),
            pl.BlockSpec((2, 64), lambda i: (0, 0)),
            pl.BlockSpec((64, 64), lambda i: (0, 0)),
        ],
        out_specs=[
            pl.BlockSpec((bn, RW), lambda i: (i, 0)),
            pl.BlockSpec(memory_space=pltpu.SMEM),
        ],
        out_shape=[
            jax.ShapeDtypeStruct((N, RW), jnp.float32),
            jax.ShapeDtypeStruct((2,), jnp.float32),
        ],
        scratch_shapes=[pltpu.VMEM((8, 128), jnp.float32)],
    )(acc1, m2, b1, r2, p64t)


def _sc2_body(tab_hbm, idx_hbm, s_hbm, out_hbm,
              ibuf0, ibuf1, srows0, drows0, srows1, drows1, stage0,
              zbuf, svec, acc, gsem0, gsem1):
    c = lax.axis_index("c")
    s = lax.axis_index("s")
    iota = lax.iota(jnp.int32, 16)
    sh1 = jnp.where(iota == 0, 1, iota)
    zidx = jnp.zeros((16,), jnp.int32)
    base128 = (c * NS + s) * (EPR // (NC * NS))

    _zero_acc(zbuf, acc, s)
    pltpu.sync_copy(s_hbm.at[c], svec)
    plsc.subcore_barrier()
    sv = svec[...]

    def make_run_edges(srows, drows, stage):
        def run_edges():
            @plsc.parallel_loop(0, SB, unroll=8)
            def _loop(e):
                s0 = srows[e, pl.ds(0, 16)]
                s1 = srows[e, pl.ds(16, 16)]
                s2 = srows[e, pl.ds(32, 16)]
                s3 = srows[e, pl.ds(48, 16)]
                m = (s0 * drows[e, pl.ds(0, 16)]
                     + s1 * drows[e, pl.ds(16, 16)]
                     + s2 * drows[e, pl.ds(32, 16)]
                     + s3 * drows[e, pl.ds(48, 16)])
                for sh in (8, 4, 2, 1):
                    m = m + _take16(m, iota ^ sh)
                ts = srows[e, pl.ds(64, 16)]
                td = drows[e, pl.ds(64, 16)]
                base = ts + _take16(td, sh1)
                sig = 1.0 / (1.0 + jnp.exp(-m))
                a0 = base * sig
                alpha = jnp.where(a0 >= 0, a0, NEG * a0)
                w = jnp.exp(alpha - sv)
                wb = _take16(w, zidx)
                stage[e, pl.ds(0, 16)] = s0 * wb
                stage[e, pl.ds(16, 16)] = s1 * wb
                stage[e, pl.ds(32, 16)] = s2 * wb
                stage[e, pl.ds(48, 16)] = s3 * wb
                stage[e, pl.ds(64, 16)] = w
        return run_edges

    eb0 = make_run_edges(srows0, drows0, stage0)
    eb1 = make_run_edges(srows1, drows1, stage0)

    def fire(sb, ibuf, srows, drows, gsem):
        _fire(idx_hbm.at[pl.ds(base128 + sb, 1)], ibuf,
              tab_hbm, srows, drows, gsem)

    fire(0, ibuf0, srows0, drows0, gsem0)
    fire(1, ibuf1, srows1, drows1, gsem1)

    def pair_body(k, _):
        _consume(ibuf0, tab_hbm, srows0, drows0, stage0, acc, gsem0, eb0)
        fire(2 * k + 2, ibuf0, srows0, drows0, gsem0)
        _consume(ibuf1, tab_hbm, srows1, drows1, stage0, acc, gsem1, eb1)
        fire(2 * k + 3, ibuf1, srows1, drows1, gsem1)
        return 0

    lax.fori_loop(0, NB2 // 2 - 1, pair_body, 0)
    _consume(ibuf0, tab_hbm, srows0, drows0, stage0, acc, gsem0, eb0)
    _consume(ibuf1, tab_hbm, srows1, drows1, stage0, acc, gsem1, eb1)
    plsc.subcore_barrier()
    pltpu.sync_copy(acc.at[pl.ds(s * ROWS_PT, ROWS_PT)],
                    out_hbm.at[c, pl.ds(s * ROWS_PT, ROWS_PT)])


_sc2_call = functools.partial(
    pl.kernel,
    out_type=jax.ShapeDtypeStruct((NC, NP, RW), jnp.float32),
    mesh=_mesh,
    compiler_params=pltpu.CompilerParams(use_tc_tiling_on_sc=False),
    scratch_types=_sc_scratch,
)(_sc2_body)


def _k3_body(acc_ref, b2_ref, out_ref):
    a0 = acc_ref[0, :, :]
    a1 = acc_ref[1, :, :]
    num = a0[:, :OUT_D] + a1[:, :OUT_D]
    den = a0[:, OUT_D:OUT_D + 1] + a1[:, OUT_D:OUT_D + 1]
    o = num * (1.0 / (den + 1e-16)) + b2_ref[...]
    m = jnp.max(o, axis=1, keepdims=True)
    z = o - m
    lse = jnp.log(jnp.sum(jnp.exp(z), axis=1, keepdims=True))
    out_ref[...] = z - lse


def _tc_final(acc2, b2):
    bn = 2000
    grid = N // bn
    return pl.pallas_call(
        _k3_body,
        grid=(grid,),
        in_specs=[
            pl.BlockSpec((2, bn, RW), lambda i: (0, i, 0)),
            pl.BlockSpec((1, OUT_D), lambda i: (0, 0)),
        ],
        out_specs=pl.BlockSpec((bn, OUT_D), lambda i: (i, 0)),
        out_shape=jax.ShapeDtypeStruct((N, OUT_D), jnp.float32),
    )(acc2, b2)


def kernel(x, edge_index, W1, att_l1, att_r1, b1, W2, att_l2, att_r2, b2):
    src, dst = edge_index[0], edge_index[1]
    mask = src != dst
    src = jnp.where(mask, src, 0)
    dst = jnp.where(mask, dst, N)
    loop = jnp.arange(N, dtype=src.dtype)
    src = jnp.concatenate([src, loop])
    dst = jnp.concatenate([dst, loop])
    npad = EP - ET
    pidx = jnp.arange(npad, dtype=jnp.int32)
    src = jnp.concatenate([src, pidx % 64])
    dst = jnp.concatenate([dst, N + 1 + (pidx % (NP - N - 1))])
    eidx = jnp.arange(EP, dtype=jnp.int32)
    dst_g = jnp.where(dst < N, dst, eidx % 64)
    idx = jnp.stack([src.reshape(EPR, 128), dst_g.reshape(EPR, 128),
                     dst.reshape(EPR, 128)], axis=1)

    jarr = jnp.arange(HEADS * HID)
    head = jarr // HID
    ch = jarr % HID
    attl_flat = att_l1.reshape(HEADS * HID)
    attr_flat = att_r1.reshape(HEADS * HID)
    onehot = (head[:, None] == jnp.arange(8)[None, :]).astype(jnp.float32)
    attl_mat = attl_flat[:, None] * onehot
    attr_mat = attr_flat[:, None] * onehot
    cols = jnp.arange(RW)[None, :]
    ms = []
    for q in range(4):
        pairmask = (jarr // 64 == q).astype(jnp.float32)
        dcol = 2 * ch + (head % 2)
        feat = pairmask[:, None] * (cols == dcol[:, None]).astype(jnp.float32)
        alc = (attl_flat * (head == 2 * q))[:, None] * (cols == 64)
        alc2 = (attl_flat * (head == 2 * q + 1))[:, None] * (cols == 65)
        arc = (attr_flat * (head == 2 * q))[:, None] * (cols == 66)
        arc2 = (attr_flat * (head == 2 * q + 1))[:, None] * (cols == 67)
        ms.append(feat + alc + alc2 + arc + arc2)
    ms = jnp.stack(ms)
    r2 = (jnp.arange(2)[:, None] == (jnp.arange(64)[None, :] // HID)
          ).astype(jnp.float32)
    jj = jnp.arange(64)
    dcol64 = 2 * (jj % 32) + jj // 32
    p64t = (dcol64[:, None] == jj[None, :]).astype(jnp.float32).T
    m2 = jnp.concatenate([
        W2, W2 @ att_l2.reshape(OUT_D, 1), W2 @ att_r2.reshape(OUT_D, 1),
        jnp.zeros((HEADS * HID, RW - OUT_D - 2), jnp.float32)], axis=1)

    tab1, s1 = _tc_layer1(x, W1, attl_mat, attr_mat, ms)
    z14 = jnp.zeros(14, jnp.float32)
    s1v = jnp.stack([
        jnp.concatenate([s1[2 * q:2 * q + 2], z14]) for q in range(4)])
    acc1 = _sc1_call(tab1, idx, s1v)

    tab2, s2 = _tc_layer2(acc1, m2, b1.reshape(1, HEADS * HID), r2, p64t)
    s2v = jnp.stack([
        jnp.concatenate([s2[0:1], jnp.zeros(15, jnp.float32)]),
        jnp.concatenate([s2[0:1], jnp.zeros(15, jnp.float32)]),
    ])
    acc2 = _sc2_call(tab2, idx, s2v)

    return _tc_final(acc2, b2.reshape(1, OUT_D))


# speedup vs baseline: 1.1204x; 1.1204x over previous
import functools

import jax
import jax.numpy as jnp
from jax import lax
from jax.experimental import pallas as pl
from jax.experimental.pallas import tpu as pltpu
from jax.experimental.pallas import tpu_sc as plsc

N = 10000
E = 320000
IN_D = 128
HID = 32
HEADS = 8
OUT_D = 64
NEG = 0.2

NC, NS, LANES = 2, 16, 16
ET = E + N
SB = 128
EPR = 2688
EP = EPR * 128
NP = 10240
RW = 80
ROWS_PT = NP // NS
ZB = 32

NB1 = EPR // NS
NB2 = EPR // (NC * NS)

_mesh = plsc.VectorSubcoreMesh(core_axis_name="c", subcore_axis_name="s")


def _take16(v, idx):
    dn = lax.GatherDimensionNumbers(
        offset_dims=(), collapsed_slice_dims=(0,), start_index_map=(0,))
    return lax.gather(v, idx[:, None], dn, (1,),
                      mode=lax.GatherScatterMode.PROMISE_IN_BOUNDS)


def _k1_body(x_ref, w1_ref, attl_ref, attr_ref, ms_ref, tab_ref, s_ref,
             vmax):
    i = pl.program_id(0)
    qq = pl.program_id(1)
    h = jnp.dot(x_ref[...], w1_ref[...], preferred_element_type=jnp.float32)
    tab_ref[...] = jnp.dot(h, ms_ref[0], preferred_element_type=jnp.float32)

    @pl.when(qq == 0)
    def _():
        al = jnp.dot(h, attl_ref[...], preferred_element_type=jnp.float32)
        ar = jnp.dot(h, attr_ref[...], preferred_element_type=jnp.float32)

        @pl.when(i == 0)
        def _():
            vmax[...] = jnp.full((8, 128), -jnp.inf, jnp.float32)

        vmax[0:1, 0:8] = jnp.maximum(
            vmax[0:1, 0:8], jnp.max(al, axis=0, keepdims=True))
        vmax[0:1, 8:16] = jnp.maximum(
            vmax[0:1, 8:16], jnp.max(ar, axis=0, keepdims=True))

    @pl.when((i == pl.num_programs(0) - 1) & (qq == pl.num_programs(1) - 1))
    def _():
        for hh in range(8):
            s_ref[hh] = jnp.maximum(vmax[0, hh] + vmax[0, 8 + hh], 0.0)
        for k in range(8, 16):
            s_ref[k] = 0.0


def _tc_layer1(x, W1, attl_mat, attr_mat, ms):
    bn = 2000
    grid = N // bn
    return pl.pallas_call(
        _k1_body,
        grid=(grid, 4),
        in_specs=[
            pl.BlockSpec((bn, IN_D), lambda i, qq: (i, 0)),
            pl.BlockSpec((IN_D, HEADS * HID), lambda i, qq: (0, 0)),
            pl.BlockSpec((HEADS * HID, 8), lambda i, qq: (0, 0)),
            pl.BlockSpec((HEADS * HID, 8), lambda i, qq: (0, 0)),
            pl.BlockSpec((1, HEADS * HID, RW), lambda i, qq: (qq, 0, 0)),
        ],
        out_specs=[
            pl.BlockSpec((bn, RW), lambda i, qq: (qq * (N // bn) + i, 0)),
            pl.BlockSpec(memory_space=pltpu.SMEM),
        ],
        out_shape=[
            jax.ShapeDtypeStruct((4 * N, RW), jnp.float32),
            jax.ShapeDtypeStruct((16,), jnp.float32),
        ],
        scratch_shapes=[pltpu.VMEM((8, 128), jnp.float32)],
    )(x, W1, attl_mat, attr_mat, ms)


def _zero_acc(zbuf, acc, s):
    zero16 = jnp.zeros((16,), jnp.float32)
    for r in range(ZB):
        for k in range(RW // 16):
            zbuf[r, pl.ds(16 * k, 16)] = zero16
    for j in range(ROWS_PT // ZB):
        pltpu.sync_copy(zbuf, acc.at[pl.ds(s * ROWS_PT + j * ZB, ZB)])


def _fire(idx_slice, ibuf, tab_hbm, srows, drows, gsem, qoff=None):
    pltpu.sync_copy(idx_slice, ibuf)
    if qoff is not None:
        for j in range(2):
            for k in range(8):
                sl = pl.ds(16 * k, 16)
                ibuf[0, j, sl] = ibuf[0, j, sl] + qoff
    pltpu.async_copy(tab_hbm.at[ibuf.at[0, 0]], srows, gsem)
    pltpu.async_copy(tab_hbm.at[ibuf.at[0, 1]], drows, gsem)


def _consume(ibuf, tab_hbm, srows, drows, stage, acc, gsem, run_edges):
    pltpu.make_async_copy(tab_hbm.at[ibuf.at[0, 0]], srows, gsem).wait()
    pltpu.make_async_copy(tab_hbm.at[ibuf.at[0, 1]], drows, gsem).wait()
    run_edges()
    pltpu.sync_copy(stage, acc.at[ibuf.at[0, 2]], add=True)


def _sc1_body(tab_hbm, idx_hbm, s_hbm, out_hbm,
              ibuf0, ibuf1, srows0, drows0, srows1, drows1, stage0,
              zbuf, svec, acc, gsem0, gsem1):
    c = lax.axis_index("c")
    s = lax.axis_index("s")
    iota = lax.iota(jnp.int32, 16)
    sh2 = jnp.where(iota < 2, iota + 2, iota)
    par = iota & 1
    base128 = s * (EPR // NS)

    for p in range(2):
        q = 2 * c + p
        qoff = q * N
        _zero_acc(zbuf, acc, s)
        pltpu.sync_copy(s_hbm.at[q], svec)
        plsc.subcore_barrier()
        sv = svec[...]

        def make_run_edges(srows, drows, stage):
            def run_edges():
                @plsc.parallel_loop(0, SB, unroll=4)
                def _loop(e):
                    s0 = srows[e, pl.ds(0, 16)]
                    s1 = srows[e, pl.ds(16, 16)]
                    s2 = srows[e, pl.ds(32, 16)]
                    s3 = srows[e, pl.ds(48, 16)]
                    m = (s0 * drows[e, pl.ds(0, 16)]
                         + s1 * drows[e, pl.ds(16, 16)]
                         + s2 * drows[e, pl.ds(32, 16)]
                         + s3 * drows[e, pl.ds(48, 16)])
                    for sh in (8, 4, 2):
                        m = m + _take16(m, iota ^ sh)
                    ts = srows[e, pl.ds(64, 16)]
                    td = drows[e, pl.ds(64, 16)]
                    base = ts + _take16(td, sh2)
                    sig = 1.0 / (1.0 + jnp.exp(-m))
                    a0 = base * sig
                    alpha = jnp.where(a0 >= 0, a0, NEG * a0)
                    w = jnp.exp(alpha - sv)
                    wmix = _take16(w, par)
                    stage[e, pl.ds(0, 16)] = s0 * wmix
                    stage[e, pl.ds(16, 16)] = s1 * wmix
                    stage[e, pl.ds(32, 16)] = s2 * wmix
                    stage[e, pl.ds(48, 16)] = s3 * wmix
                    stage[e, pl.ds(64, 16)] = w
            return run_edges

        eb0 = make_run_edges(srows0, drows0, stage0)
        eb1 = make_run_edges(srows1, drows1, stage0)

        def fire(sb, ibuf, srows, drows, gsem):
            _fire(idx_hbm.at[pl.ds(base128 + sb, 1)], ibuf,
                  tab_hbm, srows, drows, gsem, qoff=qoff)

        fire(0, ibuf0, srows0, drows0, gsem0)
        fire(1, ibuf1, srows1, drows1, gsem1)

        def pair_body(k, _):
            _consume(ibuf0, tab_hbm, srows0, drows0, stage0, acc, gsem0, eb0)
            fire(2 * k + 2, ibuf0, srows0, drows0, gsem0)
            _consume(ibuf1, tab_hbm, srows1, drows1, stage0, acc, gsem1, eb1)
            fire(2 * k + 3, ibuf1, srows1, drows1, gsem1)
            return 0

        lax.fori_loop(0, NB1 // 2 - 1, pair_body, 0)
        _consume(ibuf0, tab_hbm, srows0, drows0, stage0, acc, gsem0, eb0)
        _consume(ibuf1, tab_hbm, srows1, drows1, stage0, acc, gsem1, eb1)
        plsc.subcore_barrier()
        pltpu.sync_copy(acc.at[pl.ds(s * ROWS_PT, ROWS_PT)],
                        out_hbm.at[q, pl.ds(s * ROWS_PT, ROWS_PT)])
        plsc.subcore_barrier()


_sc_scratch = [
    pltpu.VMEM((1, 3, 128), jnp.int32),
    pltpu.VMEM((1, 3, 128), jnp.int32),
    pltpu.VMEM((SB, RW), jnp.float32),
    pltpu.VMEM((SB, RW), jnp.float32),
    pltpu.VMEM((SB, RW), jnp.float32),
    pltpu.VMEM((SB, RW), jnp.float32),
    pltpu.VMEM((SB, RW), jnp.float32),
    pltpu.VMEM((ZB, RW), jnp.float32),
    pltpu.VMEM((16,), jnp.float32),
    pltpu.VMEM_SHARED((NP, RW), jnp.float32),
    pltpu.SemaphoreType.DMA,
    pltpu.SemaphoreType.DMA,
]

_sc1_call = functools.partial(
    pl.kernel,
    out_type=jax.ShapeDtypeStruct((4, NP, RW), jnp.float32),
    mesh=_mesh,
    compiler_params=pltpu.CompilerParams(use_tc_tiling_on_sc=False),
    scratch_types=_sc_scratch,
)(_sc1_body)


def _k2_body(acc_ref, m2_ref, b1_ref, r2_ref, p64t_ref, tab_ref, s_ref,
             vmax):
    i = pl.program_id(0)
    parts = []
    for q in range(4):
        aq = acc_ref[q, :, :]
        fde = jnp.dot(aq[:, :64], p64t_ref[...],
                      preferred_element_type=jnp.float32)
        rec = jnp.dot(1.0 / (aq[:, 64:66] + 1e-16), r2_ref[...],
                      preferred_element_type=jnp.float32)
        parts.append(fde * rec)
    out1 = jnp.concatenate(parts, axis=1) + b1_ref[...]
    act = jnp.where(out1 > 0, out1, jnp.exp(jnp.minimum(out1, 0.0)) - 1.0)
    tab = jnp.dot(act, m2_ref[...], preferred_element_type=jnp.float32)
    tab_ref[...] = tab

    @pl.when(i == 0)
    def _():
        vmax[...] = jnp.full((8, 128), -jnp.inf, jnp.float32)

    vmax[0:1, 0:2] = jnp.maximum(
        vmax[0:1, 0:2], jnp.max(tab[:, 64:66], axis=0, keepdims=True))

    @pl.when(i == pl.num_programs(0) - 1)
    def _():
        s_ref[0] = jnp.maximum(vmax[0, 0] + vmax[0, 1], 0.0)
        s_ref[1] = 0.0


def _tc_layer2(acc1, m2, b1, r2, p64t):
    bn = 2000
    grid = N // bn
    return pl.pallas_call(
        _k2_body,
        grid=(grid,),
        in_specs=[
            pl.BlockSpec((4, bn, RW), lambda i: (0, i, 0)),
            pl.BlockSpec((HEADS * HID, RW), lambda i: (0, 0)),
            pl.BlockSpec((1, HEADS * HID), lambda i: (0, 0)),
            pl.BlockSpec((2, 64), lambda i: (0, 0)),
            pl.BlockSpec((64, 64), lambda i: (0, 0)),
        ],
        out_specs=[
            pl.BlockSpec((bn, RW), lambda i: (i, 0)),
            pl.BlockSpec(memory_space=pltpu.SMEM),
        ],
        out_shape=[
            jax.ShapeDtypeStruct((N, RW), jnp.float32),
            jax.ShapeDtypeStruct((2,), jnp.float32),
        ],
        scratch_shapes=[pltpu.VMEM((8, 128), jnp.float32)],
    )(acc1, m2, b1, r2, p64t)


def _sc2_body(tab_hbm, idx_hbm, s_hbm, out_hbm,
              ibuf0, ibuf1, srows0, drows0, srows1, drows1, stage0,
              zbuf, svec, acc, gsem0, gsem1):
    c = lax.axis_index("c")
    s = lax.axis_index("s")
    iota = lax.iota(jnp.int32, 16)
    sh1 = jnp.where(iota == 0, 1, iota)
    zidx = jnp.zeros((16,), jnp.int32)
    base128 = (c * NS + s) * (EPR // (NC * NS))

    _zero_acc(zbuf, acc, s)
    pltpu.sync_copy(s_hbm.at[c], svec)
    plsc.subcore_barrier()
    sv = svec[...]

    def make_run_edges(srows, drows, stage):
        def run_edges():
            @plsc.parallel_loop(0, SB, unroll=4)
            def _loop(e):
                s0 = srows[e, pl.ds(0, 16)]
                s1 = srows[e, pl.ds(16, 16)]
                s2 = srows[e, pl.ds(32, 16)]
                s3 = srows[e, pl.ds(48, 16)]
                m = (s0 * drows[e, pl.ds(0, 16)]
                     + s1 * drows[e, pl.ds(16, 16)]
                     + s2 * drows[e, pl.ds(32, 16)]
                     + s3 * drows[e, pl.ds(48, 16)])
                for sh in (8, 4, 2, 1):
                    m = m + _take16(m, iota ^ sh)
                ts = srows[e, pl.ds(64, 16)]
                td = drows[e, pl.ds(64, 16)]
                base = ts + _take16(td, sh1)
                sig = 1.0 / (1.0 + jnp.exp(-m))
                a0 = base * sig
                alpha = jnp.where(a0 >= 0, a0, NEG * a0)
                w = jnp.exp(alpha - sv)
                wb = _take16(w, zidx)
                stage[e, pl.ds(0, 16)] = s0 * wb
                stage[e, pl.ds(16, 16)] = s1 * wb
                stage[e, pl.ds(32, 16)] = s2 * wb
                stage[e, pl.ds(48, 16)] = s3 * wb
                stage[e, pl.ds(64, 16)] = w
        return run_edges

    eb0 = make_run_edges(srows0, drows0, stage0)
    eb1 = make_run_edges(srows1, drows1, stage0)

    def fire(sb, ibuf, srows, drows, gsem):
        _fire(idx_hbm.at[pl.ds(base128 + sb, 1)], ibuf,
              tab_hbm, srows, drows, gsem)

    fire(0, ibuf0, srows0, drows0, gsem0)
    fire(1, ibuf1, srows1, drows1, gsem1)

    def pair_body(k, _):
        _consume(ibuf0, tab_hbm, srows0, drows0, stage0, acc, gsem0, eb0)
        fire(2 * k + 2, ibuf0, srows0, drows0, gsem0)
        _consume(ibuf1, tab_hbm, srows1, drows1, stage0, acc, gsem1, eb1)
        fire(2 * k + 3, ibuf1, srows1, drows1, gsem1)
        return 0

    lax.fori_loop(0, NB2 // 2 - 1, pair_body, 0)
    _consume(ibuf0, tab_hbm, srows0, drows0, stage0, acc, gsem0, eb0)
    _consume(ibuf1, tab_hbm, srows1, drows1, stage0, acc, gsem1, eb1)
    plsc.subcore_barrier()
    pltpu.sync_copy(acc.at[pl.ds(s * ROWS_PT, ROWS_PT)],
                    out_hbm.at[c, pl.ds(s * ROWS_PT, ROWS_PT)])


_sc2_call = functools.partial(
    pl.kernel,
    out_type=jax.ShapeDtypeStruct((NC, NP, RW), jnp.float32),
    mesh=_mesh,
    compiler_params=pltpu.CompilerParams(use_tc_tiling_on_sc=False),
    scratch_types=_sc_scratch,
)(_sc2_body)


def _k3_body(acc_ref, b2_ref, out_ref):
    a0 = acc_ref[0, :, :]
    a1 = acc_ref[1, :, :]
    num = a0[:, :OUT_D] + a1[:, :OUT_D]
    den = a0[:, OUT_D:OUT_D + 1] + a1[:, OUT_D:OUT_D + 1]
    o = num * (1.0 / (den + 1e-16)) + b2_ref[...]
    m = jnp.max(o, axis=1, keepdims=True)
    z = o - m
    lse = jnp.log(jnp.sum(jnp.exp(z), axis=1, keepdims=True))
    out_ref[...] = z - lse


def _tc_final(acc2, b2):
    bn = 2000
    grid = N // bn
    return pl.pallas_call(
        _k3_body,
        grid=(grid,),
        in_specs=[
            pl.BlockSpec((2, bn, RW), lambda i: (0, i, 0)),
            pl.BlockSpec((1, OUT_D), lambda i: (0, 0)),
        ],
        out_specs=pl.BlockSpec((bn, OUT_D), lambda i: (i, 0)),
        out_shape=jax.ShapeDtypeStruct((N, OUT_D), jnp.float32),
    )(acc2, b2)


def kernel(x, edge_index, W1, att_l1, att_r1, b1, W2, att_l2, att_r2, b2):
    src, dst = edge_index[0], edge_index[1]
    mask = src != dst
    src = jnp.where(mask, src, 0)
    dst = jnp.where(mask, dst, N)
    loop = jnp.arange(N, dtype=src.dtype)
    src = jnp.concatenate([src, loop])
    dst = jnp.concatenate([dst, loop])
    npad = EP - ET
    pidx = jnp.arange(npad, dtype=jnp.int32)
    src = jnp.concatenate([src, pidx % 64])
    dst = jnp.concatenate([dst, N + 1 + (pidx % (NP - N - 1))])
    eidx = jnp.arange(EP, dtype=jnp.int32)
    dst_g = jnp.where(dst < N, dst, eidx % 64)
    idx = jnp.stack([src.reshape(EPR, 128), dst_g.reshape(EPR, 128),
                     dst.reshape(EPR, 128)], axis=1)

    jarr = jnp.arange(HEADS * HID)
    head = jarr // HID
    ch = jarr % HID
    attl_flat = att_l1.reshape(HEADS * HID)
    attr_flat = att_r1.reshape(HEADS * HID)
    onehot = (head[:, None] == jnp.arange(8)[None, :]).astype(jnp.float32)
    attl_mat = attl_flat[:, None] * onehot
    attr_mat = attr_flat[:, None] * onehot
    cols = jnp.arange(RW)[None, :]
    ms = []
    for q in range(4):
        pairmask = (jarr // 64 == q).astype(jnp.float32)
        dcol = 2 * ch + (head % 2)
        feat = pairmask[:, None] * (cols == dcol[:, None]).astype(jnp.float32)
        alc = (attl_flat * (head == 2 * q))[:, None] * (cols == 64)
        alc2 = (attl_flat * (head == 2 * q + 1))[:, None] * (cols == 65)
        arc = (attr_flat * (head == 2 * q))[:, None] * (cols == 66)
        arc2 = (attr_flat * (head == 2 * q + 1))[:, None] * (cols == 67)
        ms.append(feat + alc + alc2 + arc + arc2)
    ms = jnp.stack(ms)
    r2 = (jnp.arange(2)[:, None] == (jnp.arange(64)[None, :] // HID)
          ).astype(jnp.float32)
    jj = jnp.arange(64)
    dcol64 = 2 * (jj % 32) + jj // 32
    p64t = (dcol64[:, None] == jj[None, :]).astype(jnp.float32).T
    m2 = jnp.concatenate([
        W2, W2 @ att_l2.reshape(OUT_D, 1), W2 @ att_r2.reshape(OUT_D, 1),
        jnp.zeros((HEADS * HID, RW - OUT_D - 2), jnp.float32)], axis=1)

    tab1, s1 = _tc_layer1(x, W1, attl_mat, attr_mat, ms)
    z14 = jnp.zeros(14, jnp.float32)
    s1v = jnp.stack([
        jnp.concatenate([s1[2 * q:2 * q + 2], z14]) for q in range(4)])
    acc1 = _sc1_call(tab1, idx, s1v)

    tab2, s2 = _tc_layer2(acc1, m2, b1.reshape(1, HEADS * HID), r2, p64t)
    s2v = jnp.stack([
        jnp.concatenate([s2[0:1], jnp.zeros(15, jnp.float32)]),
        jnp.concatenate([s2[0:1], jnp.zeros(15, jnp.float32)]),
    ])
    acc2 = _sc2_call(tab2, idx, s2v)

    return _tc_final(acc2, b2.reshape(1, OUT_D))
